# baseline (speedup 1.0000x reference)
import functools

import jax
import jax.numpy as jnp
from jax import lax
from jax.experimental import pallas as pl
from jax.experimental.pallas import tpu as pltpu
from jax.experimental.pallas import tpu_sc as plsc

NUM_CORES = 2
NUM_SUBCORES = 16
NUM_WORKERS = NUM_CORES * NUM_SUBCORES
TOK_PER_CHUNK = 2
K = 8
G = 4

TR_BLOCK = 8192


def _transpose_kernel(xt_ref, eye_ref, out_ref):
    out_ref[...] = lax.dot_general(
        xt_ref[...], eye_ref[...], (((0,), (0,)), ((), ())),
        preferred_element_type=jnp.float32,
    )


def _table_to_row_major(table):
    v, d = table.shape
    tt = table.T
    eye = jnp.eye(d, dtype=jnp.float32)
    grid = (v + TR_BLOCK - 1) // TR_BLOCK
    return pl.pallas_call(
        _transpose_kernel,
        grid=(grid,),
        in_specs=[
            pl.BlockSpec((d, TR_BLOCK), lambda i: (0, i)),
            pl.BlockSpec((d, d), lambda i: (0, 0)),
        ],
        out_specs=pl.BlockSpec((TR_BLOCK, d), lambda i: (i, 0)),
        out_shape=jax.ShapeDtypeStruct((v, d), jnp.float32),
    )(tt, eye)


@functools.partial(jax.jit, static_argnums=(2,))
def _embedding_lookup(idx2d, table, seq):
    n_tokens = idx2d.shape[0] * TOK_PER_CHUNK
    dim = table.shape[1]
    chunk_idx = TOK_PER_CHUNK * seq
    n = idx2d.shape[0] // NUM_WORKERS
    assert n % K == 0
    mesh = plsc.VectorSubcoreMesh(core_axis_name="c", subcore_axis_name="s")

    @functools.partial(
        pl.kernel,
        mesh=mesh,
        out_type=jax.ShapeDtypeStruct((n_tokens, seq, dim), jnp.float32),
        scratch_types=[
            pltpu.VMEM((n, chunk_idx), jnp.int32),
            pltpu.VMEM((K, chunk_idx, dim), jnp.float32),
            pltpu.SemaphoreType.DMA((K,)),
            pltpu.SemaphoreType.DMA((K,)),
        ],
        compiler_params=pltpu.CompilerParams(use_tc_tiling_on_sc=False),
    )
    def emb_kernel(idx_hbm, table_hbm, out_hbm, idx_v, rows_v, gsem, wsem):
        wid = lax.axis_index("s") * NUM_CORES + lax.axis_index("c")
        chunk_base = wid * n
        tok_base = chunk_base * TOK_PER_CHUNK
        pltpu.sync_copy(idx_hbm.at[pl.ds(chunk_base, n)], idx_v)

        def fire_gather(j, b):
            pltpu.async_copy(table_hbm.at[idx_v.at[j]], rows_v.at[b], gsem.at[b])

        def wait_gather(j, b):
            pltpu.make_async_copy(
                table_hbm.at[idx_v.at[j]], rows_v.at[b], gsem.at[b]
            ).wait()

        def write_parts(j, b):
            tok = tok_base + j * TOK_PER_CHUNK
            return [
                (rows_v.at[b, pl.ds(t * seq, seq)], out_hbm.at[tok + t])
                for t in range(TOK_PER_CHUNK)
            ]

        def fire_write(j, b):
            for src, dst in write_parts(j, b):
                pltpu.async_copy(src, dst, wsem.at[b])

        def wait_write(j, b):
            for src, dst in write_parts(j, b):
                pltpu.make_async_copy(src, dst, wsem.at[b]).wait()

        for b in range(G):
            fire_gather(b, b)

        def group(g, _):
            for b in range(K):
                j = g * K + b
                bn = (b + G) % K
                wait_gather(j, b)
                fire_write(j, b)

                @pl.when(jnp.logical_and(j + G < n, j + G - K >= 0))
                def _():
                    wait_write(j + G - K, bn)

                @pl.when(j + G < n)
                def _():
                    fire_gather(j + G, bn)

            return 0

        lax.fori_loop(0, n // K, group, 0)
        for b in range(K):
            wait_write(n - K + b, b)

    return emb_kernel(idx2d, table)


def kernel(token_ids, embedding_matrix):
    n_tokens, seq = token_ids.shape
    flat = token_ids.reshape(-1).astype(jnp.int32)
    assert n_tokens % (NUM_WORKERS * TOK_PER_CHUNK) == 0
    idx2d = flat.reshape(n_tokens // TOK_PER_CHUNK, TOK_PER_CHUNK * seq)
    table_rm = _table_to_row_major(embedding_matrix)
    return _embedding_lookup(idx2d, table_rm, seq)


# speedup vs baseline: 1.0032x; 1.0032x over previous
import functools

import jax
import jax.numpy as jnp
from jax import lax
from jax.experimental import pallas as pl
from jax.experimental.pallas import tpu as pltpu
from jax.experimental.pallas import tpu_sc as plsc

NUM_CORES = 2
NUM_SUBCORES = 16
NUM_WORKERS = NUM_CORES * NUM_SUBCORES
TOK_PER_CHUNK = 2
K = 8
G = 4

TR_BLOCK = 8192


def _transpose_kernel(xt_ref, out_ref):
    out_ref[...] = xt_ref[...].T


def _table_to_row_major(table):
    v, d = table.shape
    tt = table.T
    grid = (v + TR_BLOCK - 1) // TR_BLOCK
    return pl.pallas_call(
        _transpose_kernel,
        grid=(grid,),
        in_specs=[pl.BlockSpec((d, TR_BLOCK), lambda i: (0, i))],
        out_specs=pl.BlockSpec((TR_BLOCK, d), lambda i: (i, 0)),
        out_shape=jax.ShapeDtypeStruct((v, d), jnp.float32),
    )(tt)


@functools.partial(jax.jit, static_argnums=(2,))
def _embedding_lookup(idx2d, table, seq):
    n_tokens = idx2d.shape[0] * TOK_PER_CHUNK
    dim = table.shape[1]
    chunk_idx = TOK_PER_CHUNK * seq
    n = idx2d.shape[0] // NUM_WORKERS
    assert n % K == 0
    mesh = plsc.VectorSubcoreMesh(core_axis_name="c", subcore_axis_name="s")

    @functools.partial(
        pl.kernel,
        mesh=mesh,
        out_type=jax.ShapeDtypeStruct((n_tokens, seq, dim), jnp.float32),
        scratch_types=[
            pltpu.VMEM((n, chunk_idx), jnp.int32),
            pltpu.VMEM((K, chunk_idx, dim), jnp.float32),
            pltpu.SemaphoreType.DMA((K,)),
            pltpu.SemaphoreType.DMA((K,)),
        ],
        compiler_params=pltpu.CompilerParams(use_tc_tiling_on_sc=False),
    )
    def emb_kernel(idx_hbm, table_hbm, out_hbm, idx_v, rows_v, gsem, wsem):
        wid = lax.axis_index("s") * NUM_CORES + lax.axis_index("c")
        chunk_base = wid * n
        tok_base = chunk_base * TOK_PER_CHUNK
        pltpu.sync_copy(idx_hbm.at[pl.ds(chunk_base, n)], idx_v)

        def fire_gather(j, b):
            pltpu.async_copy(table_hbm.at[idx_v.at[j]], rows_v.at[b], gsem.at[b])

        def wait_gather(j, b):
            pltpu.make_async_copy(
                table_hbm.at[idx_v.at[j]], rows_v.at[b], gsem.at[b]
            ).wait()

        def write_parts(j, b):
            tok = tok_base + j * TOK_PER_CHUNK
            return [
                (rows_v.at[b, pl.ds(t * seq, seq)], out_hbm.at[tok + t])
                for t in range(TOK_PER_CHUNK)
            ]

        def fire_write(j, b):
            for src, dst in write_parts(j, b):
                pltpu.async_copy(src, dst, wsem.at[b])

        def wait_write(j, b):
            for src, dst in write_parts(j, b):
                pltpu.make_async_copy(src, dst, wsem.at[b]).wait()

        for b in range(G):
            fire_gather(b, b)

        def group(g, _):
            for b in range(K):
                j = g * K + b
                bn = (b + G) % K
                wait_gather(j, b)
                fire_write(j, b)

                @pl.when(jnp.logical_and(j + G < n, j + G - K >= 0))
                def _():
                    wait_write(j + G - K, bn)

                @pl.when(j + G < n)
                def _():
                    fire_gather(j + G, bn)

            return 0

        lax.fori_loop(0, n // K, group, 0)
        for b in range(K):
            wait_write(n - K + b, b)

    return emb_kernel(idx2d, table)


def kernel(token_ids, embedding_matrix):
    n_tokens, seq = token_ids.shape
    flat = token_ids.reshape(-1).astype(jnp.int32)
    assert n_tokens % (NUM_WORKERS * TOK_PER_CHUNK) == 0
    idx2d = flat.reshape(n_tokens // TOK_PER_CHUNK, TOK_PER_CHUNK * seq)
    table_rm = _table_to_row_major(embedding_matrix)
    return _embedding_lookup(idx2d, table_rm, seq)


# speedup vs baseline: 1.0231x; 1.0198x over previous
import functools

import jax
import jax.numpy as jnp
from jax import lax
from jax.experimental import pallas as pl
from jax.experimental.pallas import tpu as pltpu
from jax.experimental.pallas import tpu_sc as plsc

NUM_CORES = 2
NUM_SUBCORES = 16
NUM_WORKERS = NUM_CORES * NUM_SUBCORES
TOK_PER_CHUNK = 2
K = 8
G = 4

TR_BLOCK = 32768


def _transpose_kernel(xt_ref, out_ref):
    out_ref[...] = xt_ref[...].T


def _table_to_row_major(table):
    v, d = table.shape
    tt = table.T
    grid = (v + TR_BLOCK - 1) // TR_BLOCK
    return pl.pallas_call(
        _transpose_kernel,
        grid=(grid,),
        in_specs=[pl.BlockSpec((d, TR_BLOCK), lambda i: (0, i))],
        out_specs=pl.BlockSpec((TR_BLOCK, d), lambda i: (i, 0)),
        out_shape=jax.ShapeDtypeStruct((v, d), jnp.float32),
    )(tt)


@functools.partial(jax.jit, static_argnums=(2,))
def _embedding_lookup(idx2d, table, seq):
    n_tokens = idx2d.shape[0] * TOK_PER_CHUNK
    dim = table.shape[1]
    chunk_idx = TOK_PER_CHUNK * seq
    n = idx2d.shape[0] // NUM_WORKERS
    assert n % K == 0
    mesh = plsc.VectorSubcoreMesh(core_axis_name="c", subcore_axis_name="s")

    @functools.partial(
        pl.kernel,
        mesh=mesh,
        out_type=jax.ShapeDtypeStruct((n_tokens, seq, dim), jnp.float32),
        scratch_types=[
            pltpu.VMEM((n, chunk_idx), jnp.int32),
            pltpu.VMEM((K, chunk_idx, dim), jnp.float32),
            pltpu.SemaphoreType.DMA((K,)),
            pltpu.SemaphoreType.DMA((K,)),
        ],
        compiler_params=pltpu.CompilerParams(use_tc_tiling_on_sc=False),
    )
    def emb_kernel(idx_hbm, table_hbm, out_hbm, idx_v, rows_v, gsem, wsem):
        wid = lax.axis_index("s") * NUM_CORES + lax.axis_index("c")
        chunk_base = wid * n
        tok_base = chunk_base * TOK_PER_CHUNK
        pltpu.sync_copy(idx_hbm.at[pl.ds(chunk_base, n)], idx_v)

        def fire_gather(j, b):
            pltpu.async_copy(table_hbm.at[idx_v.at[j]], rows_v.at[b], gsem.at[b])

        def wait_gather(j, b):
            pltpu.make_async_copy(
                table_hbm.at[idx_v.at[j]], rows_v.at[b], gsem.at[b]
            ).wait()

        def write_parts(j, b):
            tok = tok_base + j * TOK_PER_CHUNK
            return [
                (rows_v.at[b, pl.ds(t * seq, seq)], out_hbm.at[tok + t])
                for t in range(TOK_PER_CHUNK)
            ]

        def fire_write(j, b):
            for src, dst in write_parts(j, b):
                pltpu.async_copy(src, dst, wsem.at[b])

        def wait_write(j, b):
            for src, dst in write_parts(j, b):
                pltpu.make_async_copy(src, dst, wsem.at[b]).wait()

        for b in range(G):
            fire_gather(b, b)

        def group(g, _):
            for b in range(K):
                j = g * K + b
                bn = (b + G) % K
                wait_gather(j, b)
                fire_write(j, b)

                @pl.when(jnp.logical_and(j + G < n, j + G - K >= 0))
                def _():
                    wait_write(j + G - K, bn)

                @pl.when(j + G < n)
                def _():
                    fire_gather(j + G, bn)

            return 0

        lax.fori_loop(0, n // K, group, 0)
        for b in range(K):
            wait_write(n - K + b, b)

    return emb_kernel(idx2d, table)


def kernel(token_ids, embedding_matrix):
    n_tokens, seq = token_ids.shape
    flat = token_ids.reshape(-1).astype(jnp.int32)
    assert n_tokens % (NUM_WORKERS * TOK_PER_CHUNK) == 0
    idx2d = flat.reshape(n_tokens // TOK_PER_CHUNK, TOK_PER_CHUNK * seq)
    table_rm = _table_to_row_major(embedding_matrix)
    return _embedding_lookup(idx2d, table_rm, seq)
